# Initial kernel scaffold; baseline (speedup 1.0000x reference)
import jax
import jax.numpy as jnp
from jax.experimental import pallas as pl


def kernel(x, edge_index, W_l0, b_l0, W_r0, W_l1, b_l1, W_r1, W_l2, b_l2, W_r2, gamma1, beta1, gamma2, beta2):
    raise NotImplementedError("write your pallas kernel here")


# speedup vs baseline: 3.9404x; 3.9404x over previous
import functools

import jax
import jax.numpy as jnp
from jax import lax
from jax.experimental import pallas as pl
from jax.experimental.pallas import tpu as pltpu
from jax.experimental.pallas import tpu_sc as plsc

N_NODES = 10000
D = 128
EPS = 1e-5

NC = 2
NS = 16
NW = NC * NS
CH = 128
NCHUNK = 80
EPT = CH * NCHUNK
E_PAD = EPT * NW
NPAD = 10240
ZROWS = NPAD // NS

_mesh = plsc.VectorSubcoreMesh(
    core_axis_name="c", subcore_axis_name="s", num_cores=NC, num_subcores=NS)


@functools.partial(
    pl.kernel,
    out_type=[
        jax.ShapeDtypeStruct((NC, NPAD, D), jnp.float32),
        jax.ShapeDtypeStruct((NC, NPAD), jnp.float32),
    ],
    mesh=_mesh,
    scratch_types=[
        pltpu.VMEM((CH,), jnp.int32),
        pltpu.VMEM((CH,), jnp.int32),
        pltpu.VMEM((CH,), jnp.int32),
        pltpu.VMEM((CH,), jnp.int32),
        pltpu.VMEM((CH, D), jnp.float32),
        pltpu.VMEM((CH, D), jnp.float32),
        pltpu.VMEM((CH,), jnp.float32),
        pltpu.VMEM((ZROWS,), jnp.float32),
        pltpu.VMEM_SHARED((NPAD, D), jnp.float32),
        pltpu.VMEM_SHARED((NPAD,), jnp.float32),
        pltpu.SemaphoreType.DMA,
        pltpu.SemaphoreType.DMA,
    ],
)
def _sc_agg(h_hbm, src_hbm, dst_hbm, agg_out, deg_out,
            si0, si1, di0, di1, r0, r1, ones_b, zb, agg_sp, deg_sp,
            sem0, sem1):
    cid = lax.axis_index("c")
    sid = lax.axis_index("s")
    wid = sid * NC + cid
    base = wid * EPT

    def _zrow(k, _):
        i = k >> 3
        j = (k & 7) * 16
        r0[i, pl.ds(j, 16)] = jnp.zeros((16,), jnp.float32)
        return 0
    lax.fori_loop(0, CH * (D // 16), _zrow, 0)

    def _zdeg(k, _):
        zb[pl.ds(k * 16, 16)] = jnp.zeros((16,), jnp.float32)
        return 0
    lax.fori_loop(0, ZROWS // 16, _zdeg, 0)

    def _ones(k, _):
        ones_b[pl.ds(k * 16, 16)] = jnp.ones((16,), jnp.float32)
        return 0
    lax.fori_loop(0, CH // 16, _ones, 0)

    for k in range(ZROWS // CH):
        pltpu.sync_copy(r0, agg_sp.at[pl.ds(sid * ZROWS + k * CH, CH)])
    pltpu.sync_copy(zb, deg_sp.at[pl.ds(sid * ZROWS, ZROWS)])
    plsc.subcore_barrier()

    def load_idx(c, si, di):
        off = pl.multiple_of(base + c * CH, CH)
        pltpu.sync_copy(src_hbm.at[pl.ds(off, CH)], si)
        pltpu.sync_copy(dst_hbm.at[pl.ds(off, CH)], di)

    def gather(si, r, sem):
        return pltpu.make_async_copy(h_hbm.at[si], r, sem)

    def scat(r, di):
        pltpu.sync_copy(r, agg_sp.at[di], add=True)
        pltpu.sync_copy(ones_b, deg_sp.at[di], add=True)

    load_idx(0, si0, di0)
    gather(si0, r0, sem0).start()

    def body(g2, _):
        c1 = 2 * g2 + 1
        load_idx(c1, si1, di1)
        gather(si1, r1, sem1).start()
        gather(si0, r0, sem0).wait()
        scat(r0, di0)

        @pl.when(c1 + 1 < NCHUNK)
        def _():
            load_idx(c1 + 1, si0, di0)
            gather(si0, r0, sem0).start()

        gather(si1, r1, sem1).wait()
        scat(r1, di1)
        return 0
    lax.fori_loop(0, NCHUNK // 2, body, 0)

    plsc.subcore_barrier()
    pltpu.sync_copy(agg_sp.at[pl.ds(sid * ZROWS, ZROWS)],
                    agg_out.at[cid, pl.ds(sid * ZROWS, ZROWS)])
    pltpu.sync_copy(deg_sp.at[pl.ds(sid * ZROWS, ZROWS)],
                    deg_out.at[cid, pl.ds(sid * ZROWS, ZROWS)])


BLK = 1000
_GRID = N_NODES // BLK
_CDIMS = (((1,), (1,)), ((), ()))


def _dense_body(deg_ref, agg_ref, h_ref, wl_ref, bl_ref, wr_ref,
                out_ref, st_ref):
    dv = jnp.maximum(deg_ref[0] + deg_ref[1], 1.0)
    mean = (agg_ref[0] + agg_ref[1]) / dv
    hpre = (lax.dot_general(mean, wl_ref[...], _CDIMS,
                            preferred_element_type=jnp.float32)
            + bl_ref[...]
            + lax.dot_general(h_ref[...], wr_ref[...], _CDIMS,
                              preferred_element_type=jnp.float32))
    out_ref[...] = hpre

    @pl.when(pl.program_id(0) == 0)
    def _():
        st_ref[...] = jnp.zeros_like(st_ref)
    st_ref[0:1, :] += jnp.sum(hpre, axis=0, keepdims=True)
    st_ref[1:2, :] += jnp.sum(hpre * hpre, axis=0, keepdims=True)


_tc_dense = pl.pallas_call(
    _dense_body,
    grid=(_GRID,),
    in_specs=[
        pl.BlockSpec((NC, BLK, 1), lambda i: (0, i, 0)),
        pl.BlockSpec((NC, BLK, D), lambda i: (0, i, 0)),
        pl.BlockSpec((BLK, D), lambda i: (i, 0)),
        pl.BlockSpec((D, D), lambda i: (0, 0)),
        pl.BlockSpec((1, D), lambda i: (0, 0)),
        pl.BlockSpec((D, D), lambda i: (0, 0)),
    ],
    out_specs=[
        pl.BlockSpec((BLK, D), lambda i: (i, 0)),
        pl.BlockSpec((2, D), lambda i: (0, 0)),
    ],
    out_shape=[
        jax.ShapeDtypeStruct((N_NODES, D), jnp.float32),
        jax.ShapeDtypeStruct((2, D), jnp.float32),
    ],
)


def _bn_body(hp_ref, st_ref, g_ref, b_ref, out_ref):
    inv_n = 1.0 / N_NODES
    m = st_ref[0:1, :] * inv_n
    v = st_ref[1:2, :] * inv_n - m * m
    scale = g_ref[...] * lax.rsqrt(v + EPS)
    y = (hp_ref[...] - m) * scale + b_ref[...]
    out_ref[...] = jnp.maximum(y, 0.0)


_tc_bn = pl.pallas_call(
    _bn_body,
    grid=(_GRID,),
    in_specs=[
        pl.BlockSpec((BLK, D), lambda i: (i, 0)),
        pl.BlockSpec((2, D), lambda i: (0, 0)),
        pl.BlockSpec((1, D), lambda i: (0, 0)),
        pl.BlockSpec((1, D), lambda i: (0, 0)),
    ],
    out_specs=pl.BlockSpec((BLK, D), lambda i: (i, 0)),
    out_shape=jax.ShapeDtypeStruct((N_NODES, D), jnp.float32),
)


def kernel(x, edge_index, W_l0, b_l0, W_r0, W_l1, b_l1, W_r1,
           W_l2, b_l2, W_r2, gamma1, beta1, gamma2, beta2):
    src = edge_index[0]
    dst = edge_index[1]
    e = src.shape[0]
    pad = E_PAD - e
    src_p = jnp.concatenate([src, jnp.zeros((pad,), jnp.int32)])
    dummy = N_NODES + (jnp.arange(pad, dtype=jnp.int32) % (NPAD - N_NODES))
    dst_p = jnp.concatenate([dst, dummy])

    bl0 = jnp.reshape(b_l0, (1, D))
    bl1 = jnp.reshape(b_l1, (1, D))
    bl2 = jnp.reshape(b_l2, (1, D))
    g1 = jnp.reshape(gamma1, (1, D))
    g2 = jnp.reshape(gamma2, (1, D))
    be1 = jnp.reshape(beta1, (1, D))
    be2 = jnp.reshape(beta2, (1, D))

    agg0, deg = _sc_agg(x, src_p, dst_p)
    deg3 = deg[:, :, None]
    h1p, st1 = _tc_dense(deg3, agg0, x, W_l0, bl0, W_r0)
    h1 = _tc_bn(h1p, st1, g1, be1)
    agg1, _ = _sc_agg(h1, src_p, dst_p)
    h2p, st2 = _tc_dense(deg3, agg1, h1, W_l1, bl1, W_r1)
    h2 = _tc_bn(h2p, st2, g2, be2)
    agg2, _ = _sc_agg(h2, src_p, dst_p)
    h3, _ = _tc_dense(deg3, agg2, h2, W_l2, bl2, W_r2)
    return h3


# speedup vs baseline: 9.3977x; 2.3850x over previous
import functools

import jax
import jax.numpy as jnp
from jax import lax
from jax.experimental import pallas as pl
from jax.experimental.pallas import tpu as pltpu
from jax.experimental.pallas import tpu_sc as plsc

N_NODES = 10000
D = 128
EPS = 1e-5

NC = 2
NS = 16
NW = NC * NS
CH = 128
NCHUNK = 80
EPT = CH * NCHUNK
E_PAD = EPT * NW
NPAD = 10240
ZROWS = NPAD // NS

_mesh = plsc.VectorSubcoreMesh(
    core_axis_name="c", subcore_axis_name="s", num_cores=NC, num_subcores=NS)


@functools.partial(
    pl.kernel,
    out_type=[
        jax.ShapeDtypeStruct((NC, NPAD, D), jnp.float32),
        jax.ShapeDtypeStruct((NC, NPAD), jnp.float32),
    ],
    mesh=_mesh,
    scratch_types=[
        pltpu.VMEM((CH,), jnp.int32),
        pltpu.VMEM((CH,), jnp.int32),
        pltpu.VMEM((CH,), jnp.int32),
        pltpu.VMEM((CH,), jnp.int32),
        pltpu.VMEM((CH, D), jnp.float32),
        pltpu.VMEM((CH, D), jnp.float32),
        pltpu.VMEM((CH,), jnp.float32),
        pltpu.VMEM((ZROWS,), jnp.float32),
        pltpu.VMEM_SHARED((NPAD, D), jnp.float32),
        pltpu.VMEM_SHARED((NPAD,), jnp.float32),
        pltpu.SemaphoreType.DMA,
        pltpu.SemaphoreType.DMA,
    ],
)
def _sc_agg(h_hbm, src_hbm, dst_hbm, agg_out, deg_out,
            si0, si1, di0, di1, r0, r1, ones_b, zb, agg_sp, deg_sp,
            sem0, sem1):
    cid = lax.axis_index("c")
    sid = lax.axis_index("s")
    wid = sid * NC + cid
    base = wid * EPT

    def _zrow(k, _):
        i = k >> 3
        j = (k & 7) * 16
        r0[i, pl.ds(j, 16)] = jnp.zeros((16,), jnp.float32)
        return 0
    lax.fori_loop(0, CH * (D // 16), _zrow, 0)

    def _zdeg(k, _):
        zb[pl.ds(k * 16, 16)] = jnp.zeros((16,), jnp.float32)
        return 0
    lax.fori_loop(0, ZROWS // 16, _zdeg, 0)

    def _ones(k, _):
        ones_b[pl.ds(k * 16, 16)] = jnp.ones((16,), jnp.float32)
        return 0
    lax.fori_loop(0, CH // 16, _ones, 0)

    for k in range(ZROWS // CH):
        pltpu.sync_copy(r0, agg_sp.at[pl.ds(sid * ZROWS + k * CH, CH)])
    pltpu.sync_copy(zb, deg_sp.at[pl.ds(sid * ZROWS, ZROWS)])
    plsc.subcore_barrier()

    def load_idx(c, si, di):
        off = pl.multiple_of(base + c * CH, CH)
        pltpu.sync_copy(src_hbm.at[pl.ds(off, CH)], si)
        pltpu.sync_copy(dst_hbm.at[pl.ds(off, CH)], di)

    def gather(si, r, sem):
        return pltpu.make_async_copy(h_hbm.at[si], r, sem)

    def scat(r, di):
        pltpu.sync_copy(r, agg_sp.at[di], add=True)
        pltpu.sync_copy(ones_b, deg_sp.at[di], add=True)

    load_idx(0, si0, di0)
    gather(si0, r0, sem0).start()

    def body(g2, _):
        c1 = 2 * g2 + 1
        load_idx(c1, si1, di1)
        gather(si1, r1, sem1).start()
        gather(si0, r0, sem0).wait()
        scat(r0, di0)

        @pl.when(c1 + 1 < NCHUNK)
        def _():
            load_idx(c1 + 1, si0, di0)
            gather(si0, r0, sem0).start()

        gather(si1, r1, sem1).wait()
        scat(r1, di1)
        return 0
    lax.fori_loop(0, NCHUNK // 2, body, 0)

    plsc.subcore_barrier()
    pltpu.sync_copy(agg_sp.at[pl.ds(sid * ZROWS, ZROWS)],
                    agg_out.at[cid, pl.ds(sid * ZROWS, ZROWS)])
    pltpu.sync_copy(deg_sp.at[pl.ds(sid * ZROWS, ZROWS)],
                    deg_out.at[cid, pl.ds(sid * ZROWS, ZROWS)])


BLK = 1000
_GRID = N_NODES // BLK
_CDIMS = (((1,), (1,)), ((), ()))


def _dense_body(deg_ref, agg_ref, h_ref, wl_ref, bl_ref, wr_ref,
                out_ref, st_ref):
    dv = jnp.maximum(deg_ref[0] + deg_ref[1], 1.0)
    mean = (agg_ref[0] + agg_ref[1]) / dv
    hpre = (lax.dot_general(mean, wl_ref[...], _CDIMS,
                            preferred_element_type=jnp.float32)
            + bl_ref[...]
            + lax.dot_general(h_ref[...], wr_ref[...], _CDIMS,
                              preferred_element_type=jnp.float32))
    out_ref[...] = hpre

    @pl.when(pl.program_id(0) == 0)
    def _():
        st_ref[...] = jnp.zeros_like(st_ref)
    st_ref[0:1, :] += jnp.sum(hpre, axis=0, keepdims=True)
    st_ref[1:2, :] += jnp.sum(hpre * hpre, axis=0, keepdims=True)


_tc_dense = pl.pallas_call(
    _dense_body,
    grid=(_GRID,),
    in_specs=[
        pl.BlockSpec((NC, BLK, 1), lambda i: (0, i, 0)),
        pl.BlockSpec((NC, BLK, D), lambda i: (0, i, 0)),
        pl.BlockSpec((BLK, D), lambda i: (i, 0)),
        pl.BlockSpec((D, D), lambda i: (0, 0)),
        pl.BlockSpec((1, D), lambda i: (0, 0)),
        pl.BlockSpec((D, D), lambda i: (0, 0)),
    ],
    out_specs=[
        pl.BlockSpec((BLK, D), lambda i: (i, 0)),
        pl.BlockSpec((2, D), lambda i: (0, 0)),
    ],
    out_shape=[
        jax.ShapeDtypeStruct((N_NODES, D), jnp.float32),
        jax.ShapeDtypeStruct((2, D), jnp.float32),
    ],
)


def _bn_body(hp_ref, st_ref, g_ref, b_ref, out_ref):
    inv_n = 1.0 / N_NODES
    m = st_ref[0:1, :] * inv_n
    v = st_ref[1:2, :] * inv_n - m * m
    scale = g_ref[...] * lax.rsqrt(v + EPS)
    y = (hp_ref[...] - m) * scale + b_ref[...]
    out_ref[...] = jnp.maximum(y, 0.0)


_tc_bn = pl.pallas_call(
    _bn_body,
    grid=(_GRID,),
    in_specs=[
        pl.BlockSpec((BLK, D), lambda i: (i, 0)),
        pl.BlockSpec((2, D), lambda i: (0, 0)),
        pl.BlockSpec((1, D), lambda i: (0, 0)),
        pl.BlockSpec((1, D), lambda i: (0, 0)),
    ],
    out_specs=pl.BlockSpec((BLK, D), lambda i: (i, 0)),
    out_shape=jax.ShapeDtypeStruct((N_NODES, D), jnp.float32),
)


def kernel(x, edge_index, W_l0, b_l0, W_r0, W_l1, b_l1, W_r1,
           W_l2, b_l2, W_r2, gamma1, beta1, gamma2, beta2):
    src = edge_index[0]
    dst = edge_index[1]
    e = src.shape[0]
    ept_real = e // NW
    pad_pt = EPT - ept_real
    rot = jnp.arange(NW, dtype=jnp.int32)[:, None] * (pad_pt // NW)
    dummy_dst = N_NODES + (
        (jnp.arange(pad_pt, dtype=jnp.int32)[None, :] + rot) % pad_pt)
    dummy_src = jnp.broadcast_to(
        jnp.arange(pad_pt, dtype=jnp.int32), (NW, pad_pt))
    src_p = jnp.concatenate(
        [src.reshape(NW, ept_real), dummy_src], axis=1).reshape(E_PAD)
    dst_p = jnp.concatenate(
        [dst.reshape(NW, ept_real), dummy_dst], axis=1).reshape(E_PAD)

    bl0 = jnp.reshape(b_l0, (1, D))
    bl1 = jnp.reshape(b_l1, (1, D))
    bl2 = jnp.reshape(b_l2, (1, D))
    g1 = jnp.reshape(gamma1, (1, D))
    g2 = jnp.reshape(gamma2, (1, D))
    be1 = jnp.reshape(beta1, (1, D))
    be2 = jnp.reshape(beta2, (1, D))

    agg0, deg = _sc_agg(x, src_p, dst_p)
    deg3 = deg[:, :, None]
    h1p, st1 = _tc_dense(deg3, agg0, x, W_l0, bl0, W_r0)
    h1 = _tc_bn(h1p, st1, g1, be1)
    agg1, _ = _sc_agg(h1, src_p, dst_p)
    h2p, st2 = _tc_dense(deg3, agg1, h1, W_l1, bl1, W_r1)
    h2 = _tc_bn(h2p, st2, g2, be2)
    agg2, _ = _sc_agg(h2, src_p, dst_p)
    h3, _ = _tc_dense(deg3, agg2, h2, W_l2, bl2, W_r2)
    return h3


# speedup vs baseline: 9.4979x; 1.0107x over previous
import functools

import jax
import jax.numpy as jnp
from jax import lax
from jax.experimental import pallas as pl
from jax.experimental.pallas import tpu as pltpu
from jax.experimental.pallas import tpu_sc as plsc

N_NODES = 10000
D = 128
EPS = 1e-5

NC = 2
NS = 16
NW = NC * NS
CH = 128
NCHUNK = 80
EPT = CH * NCHUNK
E_PAD = EPT * NW
NPAD = 10240
ZROWS = NPAD // NS

_mesh = plsc.VectorSubcoreMesh(
    core_axis_name="c", subcore_axis_name="s", num_cores=NC, num_subcores=NS)


@functools.partial(
    pl.kernel,
    out_type=[
        jax.ShapeDtypeStruct((NC, NPAD, D), jnp.float32),
        jax.ShapeDtypeStruct((NC, NPAD), jnp.float32),
    ],
    mesh=_mesh,
    scratch_types=[
        pltpu.VMEM((CH,), jnp.int32),
        pltpu.VMEM((CH,), jnp.int32),
        pltpu.VMEM((CH,), jnp.int32),
        pltpu.VMEM((CH,), jnp.int32),
        pltpu.VMEM((CH, D), jnp.float32),
        pltpu.VMEM((CH, D), jnp.float32),
        pltpu.VMEM((CH,), jnp.float32),
        pltpu.VMEM((ZROWS,), jnp.float32),
        pltpu.VMEM_SHARED((NPAD, D), jnp.float32),
        pltpu.VMEM_SHARED((NPAD,), jnp.float32),
    ] + [pltpu.SemaphoreType.DMA] * 4,
)
def _sc_agg(h_hbm, src_hbm, dst_hbm, agg_out, deg_out,
            si0, si1, di0, di1, r0, r1,
            ones_b, zb, agg_sp, deg_sp,
            g0, g1, s0, s1):
    cid = lax.axis_index("c")
    sid = lax.axis_index("s")
    wid = sid * NC + cid
    base = wid * EPT

    def _zrow(k, _):
        i = k >> 3
        j = (k & 7) * 16
        r0[i, pl.ds(j, 16)] = jnp.zeros((16,), jnp.float32)
        return 0
    lax.fori_loop(0, CH * (D // 16), _zrow, 0)

    def _zdeg(k, _):
        zb[pl.ds(k * 16, 16)] = jnp.zeros((16,), jnp.float32)
        return 0
    lax.fori_loop(0, ZROWS // 16, _zdeg, 0)

    def _ones(k, _):
        ones_b[pl.ds(k * 16, 16)] = jnp.ones((16,), jnp.float32)
        return 0
    lax.fori_loop(0, CH // 16, _ones, 0)

    for k in range(ZROWS // CH):
        pltpu.sync_copy(r0, agg_sp.at[pl.ds(sid * ZROWS + k * CH, CH)])
    pltpu.sync_copy(zb, deg_sp.at[pl.ds(sid * ZROWS, ZROWS)])
    plsc.subcore_barrier()

    NB = 2
    sis = (si0, si1)
    dis = (di0, di1)
    rows = (r0, r1)
    gsem = (g0, g1)
    ssem = (s0, s1)

    def load_idx(c, b):
        off = pl.multiple_of(base + c * CH, CH)
        pltpu.sync_copy(src_hbm.at[pl.ds(off, CH)], sis[b])
        pltpu.sync_copy(dst_hbm.at[pl.ds(off, CH)], dis[b])

    def gather(b):
        return pltpu.make_async_copy(h_hbm.at[sis[b]], rows[b], gsem[b])

    def scat_start(b):
        pltpu.make_async_copy(rows[b], agg_sp.at[dis[b]], ssem[b]).start(add=True)
        pltpu.make_async_copy(ones_b, deg_sp.at[dis[b]], ssem[b]).start(add=True)

    def scat_wait(b):
        pltpu.make_async_copy(rows[b], agg_sp.at[dis[b]], ssem[b]).wait()
        pltpu.make_async_copy(ones_b, deg_sp.at[dis[b]], ssem[b]).wait()

    for b in range(NB):
        load_idx(b, b)
        gather(b).start()

    def body(g, _):
        for b in range(NB):
            gather(b).wait()
            scat_start(b)
        for b in range(NB):
            nc = (g + 1) * NB + b

            @pl.when(nc < NCHUNK)
            def _():
                scat_wait(b)
                load_idx(nc, b)
                gather(b).start()
        return 0
    lax.fori_loop(0, NCHUNK // NB, body, 0)
    for b in range(NB):
        scat_wait(b)

    plsc.subcore_barrier()
    pltpu.sync_copy(agg_sp.at[pl.ds(sid * ZROWS, ZROWS)],
                    agg_out.at[cid, pl.ds(sid * ZROWS, ZROWS)])
    pltpu.sync_copy(deg_sp.at[pl.ds(sid * ZROWS, ZROWS)],
                    deg_out.at[cid, pl.ds(sid * ZROWS, ZROWS)])


BLK = 1000
_GRID = N_NODES // BLK
_CDIMS = (((1,), (1,)), ((), ()))


def _dense_body(deg_ref, agg_ref, h_ref, wl_ref, bl_ref, wr_ref,
                out_ref, st_ref):
    dv = jnp.maximum(deg_ref[0] + deg_ref[1], 1.0)
    mean = (agg_ref[0] + agg_ref[1]) / dv
    hpre = (lax.dot_general(mean, wl_ref[...], _CDIMS,
                            preferred_element_type=jnp.float32)
            + bl_ref[...]
            + lax.dot_general(h_ref[...], wr_ref[...], _CDIMS,
                              preferred_element_type=jnp.float32))
    out_ref[...] = hpre

    @pl.when(pl.program_id(0) == 0)
    def _():
        st_ref[...] = jnp.zeros_like(st_ref)
    st_ref[0:1, :] += jnp.sum(hpre, axis=0, keepdims=True)
    st_ref[1:2, :] += jnp.sum(hpre * hpre, axis=0, keepdims=True)


_tc_dense = pl.pallas_call(
    _dense_body,
    grid=(_GRID,),
    in_specs=[
        pl.BlockSpec((NC, BLK, 1), lambda i: (0, i, 0)),
        pl.BlockSpec((NC, BLK, D), lambda i: (0, i, 0)),
        pl.BlockSpec((BLK, D), lambda i: (i, 0)),
        pl.BlockSpec((D, D), lambda i: (0, 0)),
        pl.BlockSpec((1, D), lambda i: (0, 0)),
        pl.BlockSpec((D, D), lambda i: (0, 0)),
    ],
    out_specs=[
        pl.BlockSpec((BLK, D), lambda i: (i, 0)),
        pl.BlockSpec((2, D), lambda i: (0, 0)),
    ],
    out_shape=[
        jax.ShapeDtypeStruct((N_NODES, D), jnp.float32),
        jax.ShapeDtypeStruct((2, D), jnp.float32),
    ],
)


def _bn_body(hp_ref, st_ref, g_ref, b_ref, out_ref):
    inv_n = 1.0 / N_NODES
    m = st_ref[0:1, :] * inv_n
    v = st_ref[1:2, :] * inv_n - m * m
    scale = g_ref[...] * lax.rsqrt(v + EPS)
    y = (hp_ref[...] - m) * scale + b_ref[...]
    out_ref[...] = jnp.maximum(y, 0.0)


_tc_bn = pl.pallas_call(
    _bn_body,
    grid=(_GRID,),
    in_specs=[
        pl.BlockSpec((BLK, D), lambda i: (i, 0)),
        pl.BlockSpec((2, D), lambda i: (0, 0)),
        pl.BlockSpec((1, D), lambda i: (0, 0)),
        pl.BlockSpec((1, D), lambda i: (0, 0)),
    ],
    out_specs=pl.BlockSpec((BLK, D), lambda i: (i, 0)),
    out_shape=jax.ShapeDtypeStruct((N_NODES, D), jnp.float32),
)


def kernel(x, edge_index, W_l0, b_l0, W_r0, W_l1, b_l1, W_r1,
           W_l2, b_l2, W_r2, gamma1, beta1, gamma2, beta2):
    src = edge_index[0]
    dst = edge_index[1]
    e = src.shape[0]
    ept_real = e // NW
    pad_pt = EPT - ept_real
    rot = jnp.arange(NW, dtype=jnp.int32)[:, None] * (pad_pt // NW)
    dummy_dst = N_NODES + (
        (jnp.arange(pad_pt, dtype=jnp.int32)[None, :] + rot) % pad_pt)
    dummy_src = jnp.broadcast_to(
        jnp.arange(pad_pt, dtype=jnp.int32), (NW, pad_pt))
    src_p = jnp.concatenate(
        [src.reshape(NW, ept_real), dummy_src], axis=1).reshape(E_PAD)
    dst_p = jnp.concatenate(
        [dst.reshape(NW, ept_real), dummy_dst], axis=1).reshape(E_PAD)

    bl0 = jnp.reshape(b_l0, (1, D))
    bl1 = jnp.reshape(b_l1, (1, D))
    bl2 = jnp.reshape(b_l2, (1, D))
    g1 = jnp.reshape(gamma1, (1, D))
    g2 = jnp.reshape(gamma2, (1, D))
    be1 = jnp.reshape(beta1, (1, D))
    be2 = jnp.reshape(beta2, (1, D))

    agg0, deg = _sc_agg(x, src_p, dst_p)
    deg3 = deg[:, :, None]
    h1p, st1 = _tc_dense(deg3, agg0, x, W_l0, bl0, W_r0)
    h1 = _tc_bn(h1p, st1, g1, be1)
    agg1, _ = _sc_agg(h1, src_p, dst_p)
    h2p, st2 = _tc_dense(deg3, agg1, h1, W_l1, bl1, W_r1)
    h2 = _tc_bn(h2p, st2, g2, be2)
    agg2, _ = _sc_agg(h2, src_p, dst_p)
    h3, _ = _tc_dense(deg3, agg2, h2, W_l2, bl2, W_r2)
    return h3


# speedup vs baseline: 9.7374x; 1.0252x over previous
import functools

import jax
import jax.numpy as jnp
from jax import lax
from jax.experimental import pallas as pl
from jax.experimental.pallas import tpu as pltpu
from jax.experimental.pallas import tpu_sc as plsc

N_NODES = 10000
D = 128
EPS = 1e-5

NC = 2
NS = 16
NW = NC * NS
CH = 128
NCHUNK = 80
NB = 2
NI = 8
EPT = CH * NCHUNK
E_PAD = EPT * NW
NPAD = 10240
ZROWS = NPAD // NS

_mesh = plsc.VectorSubcoreMesh(
    core_axis_name="c", subcore_axis_name="s", num_cores=NC, num_subcores=NS)


@functools.partial(
    pl.kernel,
    out_type=[
        jax.ShapeDtypeStruct((NC, NPAD, D), jnp.float32),
        jax.ShapeDtypeStruct((NC, NPAD), jnp.float32),
    ],
    mesh=_mesh,
    scratch_types=[
        pltpu.VMEM((NI, 1, CH), jnp.int32),
        pltpu.VMEM((NI, 1, CH), jnp.int32),
        pltpu.VMEM((CH, D), jnp.float32),
        pltpu.VMEM((CH, D), jnp.float32),
        pltpu.VMEM((CH,), jnp.float32),
        pltpu.VMEM((ZROWS,), jnp.float32),
        pltpu.VMEM_SHARED((NPAD, D), jnp.float32),
        pltpu.VMEM_SHARED((NPAD,), jnp.float32),
    ] + [pltpu.SemaphoreType.DMA] * 6,
)
def _sc_agg(h_hbm, src_hbm, dst_hbm, agg_out, deg_out,
            sis, dis, r0, r1, ones_b, zb, agg_sp, deg_sp,
            g0, g1, s0, s1, i0, i1):
    cid = lax.axis_index("c")
    sid = lax.axis_index("s")
    wid = sid * NC + cid
    base = wid * EPT

    def _zrow(k, _):
        i = k >> 3
        j = (k & 7) * 16
        r0[i, pl.ds(j, 16)] = jnp.zeros((16,), jnp.float32)
        return 0
    lax.fori_loop(0, CH * (D // 16), _zrow, 0)

    def _zdeg(k, _):
        zb[pl.ds(k * 16, 16)] = jnp.zeros((16,), jnp.float32)
        return 0
    lax.fori_loop(0, ZROWS // 16, _zdeg, 0)

    def _ones(k, _):
        ones_b[pl.ds(k * 16, 16)] = jnp.ones((16,), jnp.float32)
        return 0
    lax.fori_loop(0, CH // 16, _ones, 0)

    for k in range(ZROWS // CH):
        pltpu.sync_copy(r0, agg_sp.at[pl.ds(sid * ZROWS + k * CH, CH)])
    pltpu.sync_copy(zb, deg_sp.at[pl.ds(sid * ZROWS, ZROWS)])
    plsc.subcore_barrier()

    rows = (r0, r1)
    gsem = (g0, g1)
    ssem = (s0, s1)
    isem = (i0, i1)

    def idx_start(c, b):
        off = pl.multiple_of(base + c * CH, CH)
        s = lax.rem(c, NI)
        pltpu.make_async_copy(
            src_hbm.at[pl.ds(off, CH)], sis.at[s, 0], isem[b]).start()
        pltpu.make_async_copy(
            dst_hbm.at[pl.ds(off, CH)], dis.at[s, 0], isem[b]).start()

    def idx_wait(b):
        pltpu.make_async_copy(
            src_hbm.at[pl.ds(base, CH)], sis.at[0, 0], isem[b]).wait()
        pltpu.make_async_copy(
            dst_hbm.at[pl.ds(base, CH)], dis.at[0, 0], isem[b]).wait()

    def gather(c, b):
        s = lax.rem(c, NI)
        return pltpu.make_async_copy(h_hbm.at[sis.at[s, 0]], rows[b], gsem[b])

    def scat_start(c, b):
        s = lax.rem(c, NI)
        pltpu.make_async_copy(
            rows[b], agg_sp.at[dis.at[s, 0]], ssem[b]).start(add=True)
        pltpu.make_async_copy(
            ones_b, deg_sp.at[dis.at[s, 0]], ssem[b]).start(add=True)

    def scat_wait(b):
        pltpu.make_async_copy(
            rows[b], agg_sp.at[dis.at[0, 0]], ssem[b]).wait()
        pltpu.make_async_copy(
            ones_b, deg_sp.at[dis.at[0, 0]], ssem[b]).wait()

    for j in range(2 * NB):
        idx_start(j, j % NB)
    for b in range(NB):
        idx_wait(b)
        gather(b, b).start()

    def body(g, _):
        for b in range(NB):
            c = g * NB + b
            gather(c, b).wait()
            scat_start(c, b)
        for b in range(NB):
            nc = (g + 1) * NB + b

            @pl.when(nc < NCHUNK)
            def _():
                scat_wait(b)

                @pl.when(nc + NB < NCHUNK)
                def _():
                    idx_start(nc + NB, b)
                idx_wait(b)
                gather(nc, b).start()
        return 0
    lax.fori_loop(0, NCHUNK // NB, body, 0)
    for b in range(NB):
        scat_wait(b)

    plsc.subcore_barrier()
    pltpu.sync_copy(agg_sp.at[pl.ds(sid * ZROWS, ZROWS)],
                    agg_out.at[cid, pl.ds(sid * ZROWS, ZROWS)])
    pltpu.sync_copy(deg_sp.at[pl.ds(sid * ZROWS, ZROWS)],
                    deg_out.at[cid, pl.ds(sid * ZROWS, ZROWS)])


BLK = 1000
_GRID = N_NODES // BLK
_CDIMS = (((1,), (1,)), ((), ()))


def _dense_body(deg_ref, agg_ref, h_ref, wl_ref, bl_ref, wr_ref,
                out_ref, st_ref):
    dv = jnp.maximum(deg_ref[0] + deg_ref[1], 1.0)
    mean = (agg_ref[0] + agg_ref[1]) / dv
    hpre = (lax.dot_general(mean, wl_ref[...], _CDIMS,
                            preferred_element_type=jnp.float32)
            + bl_ref[...]
            + lax.dot_general(h_ref[...], wr_ref[...], _CDIMS,
                              preferred_element_type=jnp.float32))
    out_ref[...] = hpre

    @pl.when(pl.program_id(0) == 0)
    def _():
        st_ref[...] = jnp.zeros_like(st_ref)
    st_ref[0:1, :] += jnp.sum(hpre, axis=0, keepdims=True)
    st_ref[1:2, :] += jnp.sum(hpre * hpre, axis=0, keepdims=True)


_tc_dense = pl.pallas_call(
    _dense_body,
    grid=(_GRID,),
    in_specs=[
        pl.BlockSpec((NC, BLK, 1), lambda i: (0, i, 0)),
        pl.BlockSpec((NC, BLK, D), lambda i: (0, i, 0)),
        pl.BlockSpec((BLK, D), lambda i: (i, 0)),
        pl.BlockSpec((D, D), lambda i: (0, 0)),
        pl.BlockSpec((1, D), lambda i: (0, 0)),
        pl.BlockSpec((D, D), lambda i: (0, 0)),
    ],
    out_specs=[
        pl.BlockSpec((BLK, D), lambda i: (i, 0)),
        pl.BlockSpec((2, D), lambda i: (0, 0)),
    ],
    out_shape=[
        jax.ShapeDtypeStruct((N_NODES, D), jnp.float32),
        jax.ShapeDtypeStruct((2, D), jnp.float32),
    ],
)


def _bn_body(hp_ref, st_ref, g_ref, b_ref, out_ref):
    inv_n = 1.0 / N_NODES
    m = st_ref[0:1, :] * inv_n
    v = st_ref[1:2, :] * inv_n - m * m
    scale = g_ref[...] * lax.rsqrt(v + EPS)
    y = (hp_ref[...] - m) * scale + b_ref[...]
    out_ref[...] = jnp.maximum(y, 0.0)


_tc_bn = pl.pallas_call(
    _bn_body,
    grid=(_GRID,),
    in_specs=[
        pl.BlockSpec((BLK, D), lambda i: (i, 0)),
        pl.BlockSpec((2, D), lambda i: (0, 0)),
        pl.BlockSpec((1, D), lambda i: (0, 0)),
        pl.BlockSpec((1, D), lambda i: (0, 0)),
    ],
    out_specs=pl.BlockSpec((BLK, D), lambda i: (i, 0)),
    out_shape=jax.ShapeDtypeStruct((N_NODES, D), jnp.float32),
)


def kernel(x, edge_index, W_l0, b_l0, W_r0, W_l1, b_l1, W_r1,
           W_l2, b_l2, W_r2, gamma1, beta1, gamma2, beta2):
    src = edge_index[0]
    dst = edge_index[1]
    e = src.shape[0]
    ept_real = e // NW
    pad_pt = EPT - ept_real
    rot = jnp.arange(NW, dtype=jnp.int32)[:, None] * (pad_pt // NW)
    dummy_dst = N_NODES + (
        (jnp.arange(pad_pt, dtype=jnp.int32)[None, :] + rot) % pad_pt)
    dummy_src = jnp.broadcast_to(
        jnp.arange(pad_pt, dtype=jnp.int32), (NW, pad_pt))
    src_p = jnp.concatenate(
        [src.reshape(NW, ept_real), dummy_src], axis=1).reshape(E_PAD)
    dst_p = jnp.concatenate(
        [dst.reshape(NW, ept_real), dummy_dst], axis=1).reshape(E_PAD)

    bl0 = jnp.reshape(b_l0, (1, D))
    bl1 = jnp.reshape(b_l1, (1, D))
    bl2 = jnp.reshape(b_l2, (1, D))
    g1 = jnp.reshape(gamma1, (1, D))
    g2 = jnp.reshape(gamma2, (1, D))
    be1 = jnp.reshape(beta1, (1, D))
    be2 = jnp.reshape(beta2, (1, D))

    agg0, deg = _sc_agg(x, src_p, dst_p)
    deg3 = deg[:, :, None]
    h1p, st1 = _tc_dense(deg3, agg0, x, W_l0, bl0, W_r0)
    h1 = _tc_bn(h1p, st1, g1, be1)
    agg1, _ = _sc_agg(h1, src_p, dst_p)
    h2p, st2 = _tc_dense(deg3, agg1, h1, W_l1, bl1, W_r1)
    h2 = _tc_bn(h2p, st2, g2, be2)
    agg2, _ = _sc_agg(h2, src_p, dst_p)
    h3, _ = _tc_dense(deg3, agg2, h2, W_l2, bl2, W_r2)
    return h3


# speedup vs baseline: 12.1934x; 1.2522x over previous
import functools

import jax
import jax.numpy as jnp
from jax import lax
from jax.experimental import pallas as pl
from jax.experimental.pallas import tpu as pltpu
from jax.experimental.pallas import tpu_sc as plsc

N_NODES = 10000
D = 128
EPS = 1e-5

NC = 2
NS = 16
NW = NC * NS
CH = 64
NCHUNK = 160
NB = 5
NI = 16
EPT = CH * NCHUNK
E_PAD = EPT * NW
NPAD = 10240
ZROWS = NPAD // NS

_mesh = plsc.VectorSubcoreMesh(
    core_axis_name="c", subcore_axis_name="s", num_cores=NC, num_subcores=NS)


@functools.partial(
    pl.kernel,
    out_type=[
        jax.ShapeDtypeStruct((NC, NPAD, D), jnp.float32),
        jax.ShapeDtypeStruct((NC, NPAD), jnp.float32),
    ],
    mesh=_mesh,
    scratch_types=[
        pltpu.VMEM((NI, 1, CH), jnp.int32),
        pltpu.VMEM((NI, 1, CH), jnp.int32),
        pltpu.VMEM((CH, D), jnp.float32),
        pltpu.VMEM((CH, D), jnp.float32),
        pltpu.VMEM((CH, D), jnp.float32),
        pltpu.VMEM((CH, D), jnp.float32),
        pltpu.VMEM((CH, D), jnp.float32),
        pltpu.VMEM((CH,), jnp.float32),
        pltpu.VMEM((ZROWS,), jnp.float32),
        pltpu.VMEM_SHARED((NPAD, D), jnp.float32),
        pltpu.VMEM_SHARED((NPAD,), jnp.float32),
    ] + [pltpu.SemaphoreType.DMA] * 15,
)
def _sc_agg(h_hbm, src_hbm, dst_hbm, agg_out, deg_out,
            sis, dis, r0, r1, r2, r3, r4, ones_b, zb, agg_sp, deg_sp,
            g0, g1, g2, g3, g4, s0, s1, s2, s3, s4, i0, i1, i2, i3, i4):
    cid = lax.axis_index("c")
    sid = lax.axis_index("s")
    wid = sid * NC + cid
    base = wid * EPT

    def _zrow(k, _):
        i = k >> 3
        j = (k & 7) * 16
        r0[i, pl.ds(j, 16)] = jnp.zeros((16,), jnp.float32)
        return 0
    lax.fori_loop(0, CH * (D // 16), _zrow, 0)

    def _zdeg(k, _):
        zb[pl.ds(k * 16, 16)] = jnp.zeros((16,), jnp.float32)
        return 0
    lax.fori_loop(0, ZROWS // 16, _zdeg, 0)

    def _ones(k, _):
        ones_b[pl.ds(k * 16, 16)] = jnp.ones((16,), jnp.float32)
        return 0
    lax.fori_loop(0, CH // 16, _ones, 0)

    for k in range(ZROWS // CH):
        pltpu.sync_copy(r0, agg_sp.at[pl.ds(sid * ZROWS + k * CH, CH)])
    pltpu.sync_copy(zb, deg_sp.at[pl.ds(sid * ZROWS, ZROWS)])
    plsc.subcore_barrier()

    rows = (r0, r1, r2, r3, r4)
    gsem = (g0, g1, g2, g3, g4)
    ssem = (s0, s1, s2, s3, s4)
    isem = (i0, i1, i2, i3, i4)

    def idx_start(c, b):
        off = pl.multiple_of(base + c * CH, CH)
        s = lax.rem(c, NI)
        pltpu.make_async_copy(
            src_hbm.at[pl.ds(off, CH)], sis.at[s, 0], isem[b]).start()
        pltpu.make_async_copy(
            dst_hbm.at[pl.ds(off, CH)], dis.at[s, 0], isem[b]).start()

    def idx_wait(b):
        pltpu.make_async_copy(
            src_hbm.at[pl.ds(base, CH)], sis.at[0, 0], isem[b]).wait()
        pltpu.make_async_copy(
            dst_hbm.at[pl.ds(base, CH)], dis.at[0, 0], isem[b]).wait()

    def gather(c, b):
        s = lax.rem(c, NI)
        return pltpu.make_async_copy(h_hbm.at[sis.at[s, 0]], rows[b], gsem[b])

    def scat_start(c, b):
        s = lax.rem(c, NI)
        pltpu.make_async_copy(
            rows[b], agg_sp.at[dis.at[s, 0]], ssem[b]).start(add=True)
        pltpu.make_async_copy(
            ones_b, deg_sp.at[dis.at[s, 0]], ssem[b]).start(add=True)

    def scat_wait(b):
        pltpu.make_async_copy(
            rows[b], agg_sp.at[dis.at[0, 0]], ssem[b]).wait()
        pltpu.make_async_copy(
            ones_b, deg_sp.at[dis.at[0, 0]], ssem[b]).wait()

    for j in range(2 * NB):
        idx_start(j, j % NB)
    for b in range(NB):
        idx_wait(b)
        gather(b, b).start()

    def body(g, _):
        for b in range(NB):
            c = g * NB + b
            gather(c, b).wait()
            scat_start(c, b)
        for b in range(NB):
            nc = (g + 1) * NB + b

            @pl.when(nc < NCHUNK)
            def _():
                scat_wait(b)

                @pl.when(nc + NB < NCHUNK)
                def _():
                    idx_start(nc + NB, b)
                idx_wait(b)
                gather(nc, b).start()
        return 0
    lax.fori_loop(0, NCHUNK // NB, body, 0)
    for b in range(NB):
        scat_wait(b)

    plsc.subcore_barrier()
    pltpu.sync_copy(agg_sp.at[pl.ds(sid * ZROWS, ZROWS)],
                    agg_out.at[cid, pl.ds(sid * ZROWS, ZROWS)])
    pltpu.sync_copy(deg_sp.at[pl.ds(sid * ZROWS, ZROWS)],
                    deg_out.at[cid, pl.ds(sid * ZROWS, ZROWS)])


BLK = 1000
_GRID = N_NODES // BLK
_CDIMS = (((1,), (1,)), ((), ()))


def _dense_body(deg_ref, agg_ref, h_ref, wl_ref, bl_ref, wr_ref,
                out_ref, st_ref):
    dv = jnp.maximum(deg_ref[0] + deg_ref[1], 1.0)
    mean = (agg_ref[0] + agg_ref[1]) / dv
    hpre = (lax.dot_general(mean, wl_ref[...], _CDIMS,
                            preferred_element_type=jnp.float32)
            + bl_ref[...]
            + lax.dot_general(h_ref[...], wr_ref[...], _CDIMS,
                              preferred_element_type=jnp.float32))
    out_ref[...] = hpre

    @pl.when(pl.program_id(0) == 0)
    def _():
        st_ref[...] = jnp.zeros_like(st_ref)
    st_ref[0:1, :] += jnp.sum(hpre, axis=0, keepdims=True)
    st_ref[1:2, :] += jnp.sum(hpre * hpre, axis=0, keepdims=True)


_tc_dense = pl.pallas_call(
    _dense_body,
    grid=(_GRID,),
    in_specs=[
        pl.BlockSpec((NC, BLK, 1), lambda i: (0, i, 0)),
        pl.BlockSpec((NC, BLK, D), lambda i: (0, i, 0)),
        pl.BlockSpec((BLK, D), lambda i: (i, 0)),
        pl.BlockSpec((D, D), lambda i: (0, 0)),
        pl.BlockSpec((1, D), lambda i: (0, 0)),
        pl.BlockSpec((D, D), lambda i: (0, 0)),
    ],
    out_specs=[
        pl.BlockSpec((BLK, D), lambda i: (i, 0)),
        pl.BlockSpec((2, D), lambda i: (0, 0)),
    ],
    out_shape=[
        jax.ShapeDtypeStruct((N_NODES, D), jnp.float32),
        jax.ShapeDtypeStruct((2, D), jnp.float32),
    ],
)


def _bn_body(hp_ref, st_ref, g_ref, b_ref, out_ref):
    inv_n = 1.0 / N_NODES
    m = st_ref[0:1, :] * inv_n
    v = st_ref[1:2, :] * inv_n - m * m
    scale = g_ref[...] * lax.rsqrt(v + EPS)
    y = (hp_ref[...] - m) * scale + b_ref[...]
    out_ref[...] = jnp.maximum(y, 0.0)


_tc_bn = pl.pallas_call(
    _bn_body,
    grid=(_GRID,),
    in_specs=[
        pl.BlockSpec((BLK, D), lambda i: (i, 0)),
        pl.BlockSpec((2, D), lambda i: (0, 0)),
        pl.BlockSpec((1, D), lambda i: (0, 0)),
        pl.BlockSpec((1, D), lambda i: (0, 0)),
    ],
    out_specs=pl.BlockSpec((BLK, D), lambda i: (i, 0)),
    out_shape=jax.ShapeDtypeStruct((N_NODES, D), jnp.float32),
)


def kernel(x, edge_index, W_l0, b_l0, W_r0, W_l1, b_l1, W_r1,
           W_l2, b_l2, W_r2, gamma1, beta1, gamma2, beta2):
    src = edge_index[0]
    dst = edge_index[1]
    e = src.shape[0]
    ept_real = e // NW
    pad_pt = EPT - ept_real
    rot = jnp.arange(NW, dtype=jnp.int32)[:, None] * (pad_pt // NW)
    dummy_dst = N_NODES + (
        (jnp.arange(pad_pt, dtype=jnp.int32)[None, :] + rot) % pad_pt)
    dummy_src = jnp.broadcast_to(
        jnp.arange(pad_pt, dtype=jnp.int32), (NW, pad_pt))
    src_p = jnp.concatenate(
        [src.reshape(NW, ept_real), dummy_src], axis=1).reshape(E_PAD)
    dst_p = jnp.concatenate(
        [dst.reshape(NW, ept_real), dummy_dst], axis=1).reshape(E_PAD)

    bl0 = jnp.reshape(b_l0, (1, D))
    bl1 = jnp.reshape(b_l1, (1, D))
    bl2 = jnp.reshape(b_l2, (1, D))
    g1 = jnp.reshape(gamma1, (1, D))
    g2 = jnp.reshape(gamma2, (1, D))
    be1 = jnp.reshape(beta1, (1, D))
    be2 = jnp.reshape(beta2, (1, D))

    agg0, deg = _sc_agg(x, src_p, dst_p)
    deg3 = deg[:, :, None]
    h1p, st1 = _tc_dense(deg3, agg0, x, W_l0, bl0, W_r0)
    h1 = _tc_bn(h1p, st1, g1, be1)
    agg1, _ = _sc_agg(h1, src_p, dst_p)
    h2p, st2 = _tc_dense(deg3, agg1, h1, W_l1, bl1, W_r1)
    h2 = _tc_bn(h2p, st2, g2, be2)
    agg2, _ = _sc_agg(h2, src_p, dst_p)
    h3, _ = _tc_dense(deg3, agg2, h2, W_l2, bl2, W_r2)
    return h3
